# baseline (speedup 1.0000x reference)
import jax
import jax.numpy as jnp
from jax import lax
from jax.experimental import pallas as pl
from jax.experimental.pallas import tpu as pltpu
from jax.experimental.pallas import tpu_sc as plsc

N_TX, N_USER, N_DEV = 6000, 3000, 1000
N = N_TX + N_USER + N_DEV
E = 320000
D = 128
H = 128
R = 4
C = 2

NC = 2
NS = 16
CB = 128
NPAD = N + 112
NROWS_T = NPAD // NS
CHUNKS_PER_TILE = 158
E_PAD = CHUNKS_PER_TILE * NS * CB
ZR = 640


def _sc_agg_build(with_cnt: bool):
  mesh = plsc.VectorSubcoreMesh(
      core_axis_name="c", subcore_axis_name="s", num_cores=NC, num_subcores=NS
  )
  out_type = [jax.ShapeDtypeStruct((R * NPAD, D), jnp.float32)]
  if with_cnt:
    out_type.append(jax.ShapeDtypeStruct((R * NPAD, D), jnp.float32))
  scratch = [
      pltpu.VMEM_SHARED((NPAD, D), jnp.float32),
      pltpu.VMEM((CB,), jnp.int32),
      pltpu.VMEM((CB,), jnp.int32),
      pltpu.VMEM((3 * CB,), jnp.int32),
      pltpu.VMEM((CB,), jnp.int32),
      pltpu.VMEM((CB,), jnp.int32),
      pltpu.VMEM((CB, D), jnp.float32),
      pltpu.VMEM((CB, D), jnp.float32),
      pltpu.SemaphoreType.DMA,
      pltpu.SemaphoreType.DMA,
  ]

  def body(x_hbm, edata_hbm, zero_hbm, ones_hbm, *rest):
    if with_cnt:
      (acc_hbm, cnt_hbm, acc_sh, srcb_a, srcb_b, ebuf, dridx_a, dridx_b,
       rows_a, rows_b, sem_a, sem_b) = rest
    else:
      (acc_hbm, acc_sh, srcb_a, srcb_b, ebuf, dridx_a, dridx_b,
       rows_a, rows_b, sem_a, sem_b) = rest
    srcbs, dridxs, rowss, sems = (
        (srcb_a, srcb_b), (dridx_a, dridx_b), (rows_a, rows_b), (sem_a, sem_b))
    c = lax.axis_index("c")
    s = lax.axis_index("s")
    row0 = s * NROWS_T
    dummy = N + s
    rchunks = [(k * CB, CB) for k in range(NROWS_T // CB)]
    if NROWS_T % CB:
      rchunks.append((NROWS_T - NROWS_T % CB, NROWS_T % CB))

    npass = 4 if with_cnt else 2
    for p in range(npass):
      is_cnt = p >= 2
      rel = 2 * c + (p - 2 if is_cnt else p)
      pltpu.sync_copy(zero_hbm.at[pl.ds(0, CB)], rows_a)
      for off, ln in rchunks:
        pltpu.sync_copy(rows_a.at[pl.ds(0, ln)], acc_sh.at[pl.ds(row0 + off, ln)])
      if is_cnt:
        pltpu.sync_copy(ones_hbm, rows_a)
        pltpu.sync_copy(ones_hbm, rows_b)
      plsc.subcore_barrier()

      def chunk_pair(k, carry):
        descs = [None, None]
        for t in range(2):
          gid = s * CHUNKS_PER_TILE + 2 * k + t
          pltpu.sync_copy(edata_hbm.at[pl.ds(gid * 3 * CB, 3 * CB)], ebuf)
          for i in range(CB // 16):
            sl = pl.ds(i * 16, 16)
            m = ebuf[pl.ds(2 * CB + i * 16, 16)] == rel
            dridxs[t][sl] = jnp.where(m, ebuf[pl.ds(CB + i * 16, 16)], dummy)
            if not is_cnt:
              srcbs[t][sl] = ebuf[sl]
          if not is_cnt:
            descs[t] = pltpu.async_copy(x_hbm.at[srcbs[t]], rowss[t], sems[t])
        for t in range(2):
          if not is_cnt:
            descs[t].wait()
          pltpu.sync_copy(rowss[t], acc_sh.at[dridxs[t]], add=True)
        return carry

      lax.fori_loop(0, CHUNKS_PER_TILE // 2, chunk_pair, 0)
      plsc.subcore_barrier()
      dst_out = cnt_hbm if is_cnt else acc_hbm
      for off, ln in rchunks:
        pltpu.sync_copy(acc_sh.at[pl.ds(row0 + off, ln)], rows_a.at[pl.ds(0, ln)])
        pltpu.sync_copy(
            rows_a.at[pl.ds(0, ln)], dst_out.at[pl.ds(rel * NPAD + row0 + off, ln)]
        )
      plsc.subcore_barrier()

  return pl.kernel(
      body,
      out_type=tuple(out_type) if with_cnt else out_type[0],
      mesh=mesh,
      scratch_types=scratch,
      name="sc_rgcn_agg_cnt" if with_cnt else "sc_rgcn_agg",
  )


_sc_agg_cnt = _sc_agg_build(True)
_sc_agg = _sc_agg_build(False)


def _tc_layer0(x, acc, cnt, w_root, w_rel, shift, nrows, tr=1000):
  grid = (nrows // tr,)

  def body(x_ref, acc_ref, cnt_ref, wroot_ref, wrel_ref, shift_ref, o_ref):
    m = jnp.dot(x_ref[...], wroot_ref[...], preferred_element_type=jnp.float32)
    for r in range(R):
      nb = acc_ref[r] / jnp.maximum(cnt_ref[r][:, 0:1], 1.0)
      m = m + jnp.dot(nb, wrel_ref[r], preferred_element_type=jnp.float32)
    o_ref[...] = jnp.maximum(m + shift_ref[...], 0.0)

  return pl.pallas_call(
      body,
      grid=grid,
      in_specs=[
          pl.BlockSpec((tr, D), lambda i: (i, 0)),
          pl.BlockSpec((R, tr, D), lambda i: (0, i, 0)),
          pl.BlockSpec((R, tr, D), lambda i: (0, i, 0)),
          pl.BlockSpec((D, H), lambda i: (0, 0)),
          pl.BlockSpec((R, D, H), lambda i: (0, 0, 0)),
          pl.BlockSpec((1, H), lambda i: (0, 0)),
      ],
      out_specs=pl.BlockSpec((tr, H), lambda i: (i, 0)),
      out_shape=jax.ShapeDtypeStruct((nrows, H), jnp.float32),
      name="tc_rgcn_layer0",
  )(x, acc, cnt, w_root, w_rel, shift)


def _tc_layer1(x, acc, cnt, w_root, w_rel, shift, wc_pad, bc_pad, nrows, tr=1000):
  grid = (nrows // tr,)

  def body(x_ref, acc_ref, cnt_ref, wroot_ref, wrel_ref, shift_ref, wc_ref,
           bc_ref, o_ref):
    m = jnp.dot(x_ref[...], wroot_ref[...], preferred_element_type=jnp.float32)
    for r in range(R):
      nb = acc_ref[r] / jnp.maximum(cnt_ref[r][:, 0:1], 1.0)
      m = m + jnp.dot(nb, wrel_ref[r], preferred_element_type=jnp.float32)
    z = jnp.maximum(m + shift_ref[...], 0.0)
    o_ref[...] = jnp.dot(z, wc_ref[...], preferred_element_type=jnp.float32) + bc_ref[...]

  return pl.pallas_call(
      body,
      grid=grid,
      in_specs=[
          pl.BlockSpec((tr, D), lambda i: (i, 0)),
          pl.BlockSpec((R, tr, D), lambda i: (0, i, 0)),
          pl.BlockSpec((R, tr, D), lambda i: (0, i, 0)),
          pl.BlockSpec((D, H), lambda i: (0, 0)),
          pl.BlockSpec((R, D, H), lambda i: (0, 0, 0)),
          pl.BlockSpec((1, H), lambda i: (0, 0)),
          pl.BlockSpec((H, 128), lambda i: (0, 0)),
          pl.BlockSpec((1, 128), lambda i: (0, 0)),
      ],
      out_specs=pl.BlockSpec((tr, 128), lambda i: (i, 0)),
      out_shape=jax.ShapeDtypeStruct((nrows, 128), jnp.float32),
      name="tc_rgcn_layer1_cls",
  )(x, acc, cnt, w_root, w_rel, shift, wc_pad, bc_pad)


def kernel(x_transaction, edge_index, edge_type, emb_user, emb_device,
           W_rel0, W_root0, b0, W_rel1, W_root1, b1,
           gamma0, beta0, rm0, rv0, gamma1, beta1, rm1, rv1, Wc, bc):
  x = jnp.concatenate([x_transaction, emb_user, emb_device], axis=0)

  npad_e = E_PAD - E
  src = jnp.concatenate([edge_index[0], jnp.zeros((npad_e,), jnp.int32)])
  dst = jnp.concatenate([edge_index[1], jnp.zeros((npad_e,), jnp.int32)])
  et = jnp.concatenate([edge_type, jnp.full((npad_e,), -1, jnp.int32)])
  edata = jnp.stack(
      [src.reshape(-1, CB), dst.reshape(-1, CB), et.reshape(-1, CB)], axis=1
  ).reshape(-1)
  zeros_d = jnp.zeros((ZR, D), jnp.float32)
  ones_d = jnp.ones((CB, D), jnp.float32)

  scale0 = gamma0 / jnp.sqrt(rv0 + 1e-5)
  shift0 = ((b0 - rm0) * scale0 + beta0)[None, :]
  w_root0 = W_root0 * scale0[None, :]
  w_rel0 = W_rel0 * scale0[None, None, :]
  scale1 = gamma1 / jnp.sqrt(rv1 + 1e-5)
  shift1 = ((b1 - rm1) * scale1 + beta1)[None, :]
  w_root1 = W_root1 * scale1[None, :]
  w_rel1 = W_rel1 * scale1[None, None, :]
  wc_pad = jnp.zeros((H, 128), jnp.float32).at[:, :C].set(Wc)
  bc_pad = jnp.zeros((1, 128), jnp.float32).at[0, :C].set(bc)

  acc0, cnt = _sc_agg_cnt(x, edata, zeros_d, ones_d)
  acc0 = acc0.reshape(R, NPAD, D)
  cnt = cnt.reshape(R, NPAD, D)
  h = _tc_layer0(x, acc0, cnt, w_root0, w_rel0, shift0, N)
  acc1 = _sc_agg(h, edata, zeros_d, ones_d).reshape(R, NPAD, D)
  out = _tc_layer1(h, acc1, cnt, w_root1, w_rel1, shift1, wc_pad, bc_pad, N_TX)
  return out[:, :C]


# speedup vs baseline: 1.0747x; 1.0747x over previous
import jax
import jax.numpy as jnp
from jax import lax
from jax.experimental import pallas as pl
from jax.experimental.pallas import tpu as pltpu
from jax.experimental.pallas import tpu_sc as plsc

N_TX, N_USER, N_DEV = 6000, 3000, 1000
N = N_TX + N_USER + N_DEV
E = 320000
D = 128
H = 128
R = 4
C = 2

NC = 2
NS = 16
CB = 128
NPAD = N + 112
NROWS_T = NPAD // NS
CHUNKS_PER_TILE = -(-E // (NS * CB))
E_PAD = CHUNKS_PER_TILE * NS * CB
ZR = 640


def _sc_agg_build(with_cnt: bool):
  mesh = plsc.VectorSubcoreMesh(
      core_axis_name="c", subcore_axis_name="s", num_cores=NC, num_subcores=NS
  )
  out_type = [jax.ShapeDtypeStruct((R * NPAD, D), jnp.float32)]
  if with_cnt:
    out_type.append(jax.ShapeDtypeStruct((R * NPAD, D), jnp.float32))
  scratch = [
      pltpu.VMEM_SHARED((NPAD, D), jnp.float32),
      pltpu.VMEM((CB,), jnp.int32),
      pltpu.VMEM((3 * CB,), jnp.int32),
      pltpu.VMEM((CB,), jnp.int32),
      pltpu.VMEM((CB, D), jnp.float32),
      pltpu.SemaphoreType.DMA,
  ]

  def body(x_hbm, edata_hbm, zero_hbm, ones_hbm, *rest):
    if with_cnt:
      acc_hbm, cnt_hbm, acc_sh, srcb, ebuf, dridx, rows, sem = rest
    else:
      acc_hbm, acc_sh, srcb, ebuf, dridx, rows, sem = rest
    c = lax.axis_index("c")
    s = lax.axis_index("s")
    row0 = s * NROWS_T
    dummy = N + s
    rchunks = [(k * CB, CB) for k in range(NROWS_T // CB)]
    if NROWS_T % CB:
      rchunks.append((NROWS_T - NROWS_T % CB, NROWS_T % CB))

    npass = 4 if with_cnt else 2
    for p in range(npass):
      is_cnt = p >= 2
      rel = 2 * c + (p - 2 if is_cnt else p)
      pltpu.sync_copy(zero_hbm.at[pl.ds(0, CB)], rows)
      for off, ln in rchunks:
        pltpu.sync_copy(rows.at[pl.ds(0, ln)], acc_sh.at[pl.ds(row0 + off, ln)])
      if is_cnt:
        pltpu.sync_copy(ones_hbm, rows)
      plsc.subcore_barrier()

      def chunk_body(j, carry):
        gid = s * CHUNKS_PER_TILE + j
        pltpu.sync_copy(edata_hbm.at[pl.ds(gid * 3 * CB, 3 * CB)], ebuf)
        for i in range(CB // 16):
          sl = pl.ds(i * 16, 16)
          m = ebuf[pl.ds(2 * CB + i * 16, 16)] == rel
          dridx[sl] = jnp.where(m, ebuf[pl.ds(CB + i * 16, 16)], dummy)
          if not is_cnt:
            srcb[sl] = ebuf[sl]
        if not is_cnt:
          pltpu.async_copy(x_hbm.at[srcb], rows, sem).wait()
        pltpu.sync_copy(rows, acc_sh.at[dridx], add=True)
        return carry

      lax.fori_loop(0, CHUNKS_PER_TILE, chunk_body, 0)
      plsc.subcore_barrier()
      dst_out = cnt_hbm if is_cnt else acc_hbm
      for off, ln in rchunks:
        pltpu.sync_copy(acc_sh.at[pl.ds(row0 + off, ln)], rows.at[pl.ds(0, ln)])
        pltpu.sync_copy(
            rows.at[pl.ds(0, ln)], dst_out.at[pl.ds(rel * NPAD + row0 + off, ln)]
        )
      plsc.subcore_barrier()

  return pl.kernel(
      body,
      out_type=tuple(out_type) if with_cnt else out_type[0],
      mesh=mesh,
      scratch_types=scratch,
      name="sc_rgcn_agg_cnt" if with_cnt else "sc_rgcn_agg",
  )


_sc_agg_cnt = _sc_agg_build(True)
_sc_agg = _sc_agg_build(False)


def _tc_layer0(x, acc, cnt, w_root, w_rel, shift, nrows, tr=1000):
  grid = (nrows // tr,)

  def body(x_ref, acc_ref, cnt_ref, wroot_ref, wrel_ref, shift_ref, o_ref):
    m = jnp.dot(x_ref[...], wroot_ref[...], preferred_element_type=jnp.float32)
    for r in range(R):
      nb = acc_ref[r] / jnp.maximum(cnt_ref[r][:, 0:1], 1.0)
      m = m + jnp.dot(nb, wrel_ref[r], preferred_element_type=jnp.float32)
    o_ref[...] = jnp.maximum(m + shift_ref[...], 0.0)

  return pl.pallas_call(
      body,
      grid=grid,
      in_specs=[
          pl.BlockSpec((tr, D), lambda i: (i, 0)),
          pl.BlockSpec((R, tr, D), lambda i: (0, i, 0)),
          pl.BlockSpec((R, tr, D), lambda i: (0, i, 0)),
          pl.BlockSpec((D, H), lambda i: (0, 0)),
          pl.BlockSpec((R, D, H), lambda i: (0, 0, 0)),
          pl.BlockSpec((1, H), lambda i: (0, 0)),
      ],
      out_specs=pl.BlockSpec((tr, H), lambda i: (i, 0)),
      out_shape=jax.ShapeDtypeStruct((nrows, H), jnp.float32),
      name="tc_rgcn_layer0",
  )(x, acc, cnt, w_root, w_rel, shift)


def _tc_layer1(x, acc, cnt, w_root, w_rel, shift, wc_pad, bc_pad, nrows, tr=1000):
  grid = (nrows // tr,)

  def body(x_ref, acc_ref, cnt_ref, wroot_ref, wrel_ref, shift_ref, wc_ref,
           bc_ref, o_ref):
    m = jnp.dot(x_ref[...], wroot_ref[...], preferred_element_type=jnp.float32)
    for r in range(R):
      nb = acc_ref[r] / jnp.maximum(cnt_ref[r][:, 0:1], 1.0)
      m = m + jnp.dot(nb, wrel_ref[r], preferred_element_type=jnp.float32)
    z = jnp.maximum(m + shift_ref[...], 0.0)
    o_ref[...] = jnp.dot(z, wc_ref[...], preferred_element_type=jnp.float32) + bc_ref[...]

  return pl.pallas_call(
      body,
      grid=grid,
      in_specs=[
          pl.BlockSpec((tr, D), lambda i: (i, 0)),
          pl.BlockSpec((R, tr, D), lambda i: (0, i, 0)),
          pl.BlockSpec((R, tr, D), lambda i: (0, i, 0)),
          pl.BlockSpec((D, H), lambda i: (0, 0)),
          pl.BlockSpec((R, D, H), lambda i: (0, 0, 0)),
          pl.BlockSpec((1, H), lambda i: (0, 0)),
          pl.BlockSpec((H, 128), lambda i: (0, 0)),
          pl.BlockSpec((1, 128), lambda i: (0, 0)),
      ],
      out_specs=pl.BlockSpec((tr, 128), lambda i: (i, 0)),
      out_shape=jax.ShapeDtypeStruct((nrows, 128), jnp.float32),
      name="tc_rgcn_layer1_cls",
  )(x, acc, cnt, w_root, w_rel, shift, wc_pad, bc_pad)


def kernel(x_transaction, edge_index, edge_type, emb_user, emb_device,
           W_rel0, W_root0, b0, W_rel1, W_root1, b1,
           gamma0, beta0, rm0, rv0, gamma1, beta1, rm1, rv1, Wc, bc):
  x = jnp.concatenate([x_transaction, emb_user, emb_device], axis=0)

  npad_e = E_PAD - E
  src = jnp.concatenate([edge_index[0], jnp.zeros((npad_e,), jnp.int32)])
  dst = jnp.concatenate([edge_index[1], jnp.zeros((npad_e,), jnp.int32)])
  et = jnp.concatenate([edge_type, jnp.full((npad_e,), -1, jnp.int32)])
  edata = jnp.stack(
      [src.reshape(-1, CB), dst.reshape(-1, CB), et.reshape(-1, CB)], axis=1
  ).reshape(-1)
  zeros_d = jnp.zeros((ZR, D), jnp.float32)
  ones_d = jnp.ones((CB, D), jnp.float32)

  scale0 = gamma0 / jnp.sqrt(rv0 + 1e-5)
  shift0 = ((b0 - rm0) * scale0 + beta0)[None, :]
  w_root0 = W_root0 * scale0[None, :]
  w_rel0 = W_rel0 * scale0[None, None, :]
  scale1 = gamma1 / jnp.sqrt(rv1 + 1e-5)
  shift1 = ((b1 - rm1) * scale1 + beta1)[None, :]
  w_root1 = W_root1 * scale1[None, :]
  w_rel1 = W_rel1 * scale1[None, None, :]
  wc_pad = jnp.zeros((H, 128), jnp.float32).at[:, :C].set(Wc)
  bc_pad = jnp.zeros((1, 128), jnp.float32).at[0, :C].set(bc)

  acc0, cnt = _sc_agg_cnt(x, edata, zeros_d, ones_d)
  acc0 = acc0.reshape(R, NPAD, D)
  cnt = cnt.reshape(R, NPAD, D)
  h = _tc_layer0(x, acc0, cnt, w_root0, w_rel0, shift0, N)
  acc1 = _sc_agg(h, edata, zeros_d, ones_d).reshape(R, NPAD, D)
  out = _tc_layer1(h, acc1, cnt, w_root1, w_rel1, shift1, wc_pad, bc_pad, N_TX)
  return out[:, :C]
